# baseline (speedup 1.0000x reference)
import jax
import jax.numpy as jnp
from jax import lax
from jax.experimental import pallas as pl
from jax.experimental.pallas import tpu as pltpu
from jax.experimental.pallas import tpu_sc as plsc

N_NODES = 10000
N_PAD = 10240
N_EDGES = 160000
D_IN = 256
WIDTH = 1024
D_OUT = 256
HALF = 128

NC = 2
NS = 16
L = 16

EDGES_PER_TILE = N_EDGES // NS
CHUNK_A = 2000
K = 80
CHUNK_C = 2000
SLAB = N_PAD // NS

ROW_BLK = 1024
GRID_N = N_PAD // ROW_BLK


def _degrees_body(ei_ref, cnt_out_ref, cnt_v, idx_v, sem):
    c = lax.axis_index("c")
    s = lax.axis_index("s")
    zeros16 = jnp.zeros((L,), jnp.float32)
    ones16 = jnp.ones((L,), jnp.float32)

    base = c * N_EDGES + s * EDGES_PER_TILE
    idesc = pltpu.async_copy(ei_ref.at[pl.ds(base, EDGES_PER_TILE)], idx_v,
                             sem)

    def zero_body(r, _):
        for j in range(HALF // L):
            cnt_v[r, pl.ds(j * L, L)] = zeros16
        return _

    lax.fori_loop(0, N_PAD // HALF, zero_body, None)
    idesc.wait()

    def scat_body(u, _):
        for k in range(K // L):
            idx = idx_v[pl.ds(u * K + k * L, L)]
            row = lax.shift_right_logical(idx, 7)
            col = jnp.bitwise_and(idx, 127)
            plsc.addupdate_scatter(cnt_v, [row, col], ones16)
        return _

    lax.fori_loop(0, EDGES_PER_TILE // K, scat_body, None)
    pltpu.sync_copy(cnt_v, cnt_out_ref.at[c, s])


def _degrees(ei_flat):
    return pl.kernel(
        _degrees_body,
        out_type=jax.ShapeDtypeStruct((NC, NS, N_PAD // HALF, HALF),
                                      jnp.float32),
        mesh=plsc.VectorSubcoreMesh(core_axis_name="c", subcore_axis_name="s"),
        compiler_params=pltpu.CompilerParams(needs_layout_passes=False),
        scratch_types=[
            pltpu.VMEM((N_PAD // HALF, HALF), jnp.float32),
            pltpu.VMEM((EDGES_PER_TILE,), jnp.int32),
            pltpu.SemaphoreType.DMA,
        ],
    )(ei_flat)


def _scale_body(x_ref, csrc_ref, cdst_ref, y_ref, ndst_ref):
    csum = jnp.sum(csrc_ref[...], axis=0)
    dsum = jnp.sum(cdst_ref[...], axis=0)
    nsrc = lax.rsqrt(jnp.maximum(csum, 1.0))
    ndst_ref[...] = lax.rsqrt(jnp.maximum(dsum, 1.0))[:, None]
    y_ref[0] = x_ref[:, :HALF] * nsrc[:, None]
    y_ref[1] = x_ref[:, HALF:] * nsrc[:, None]


def _scale(xp, cnt_part):
    return pl.pallas_call(
        _scale_body,
        grid=(GRID_N,),
        in_specs=[
            pl.BlockSpec((ROW_BLK, D_IN), lambda i: (i, 0)),
            pl.BlockSpec((NS, ROW_BLK), lambda i: (0, i)),
            pl.BlockSpec((NS, ROW_BLK), lambda i: (1, i)),
        ],
        out_specs=[
            pl.BlockSpec((2, ROW_BLK, HALF), lambda i: (0, i, 0)),
            pl.BlockSpec((ROW_BLK, 1), lambda i: (i, 0)),
        ],
        out_shape=[
            jax.ShapeDtypeStruct((2, N_PAD, HALF), jnp.float32),
            jax.ShapeDtypeStruct((N_PAD, 1), jnp.float32),
        ],
    )(xp, cnt_part, cnt_part)


NBUF = 3
N_CHUNKS = EDGES_PER_TILE // CHUNK_C
GROUPS_PER_CHUNK = CHUNK_C // K
N_GROUPS = EDGES_PER_TILE // K


def _edge_body(y2_ref, ei_ref, ndst_ref, agg_ref, tpart_ref,
               agg_sh, zero_v, srce0_v, dste0_v, srce1_v, dste1_v,
               sem0, sem1, sem2, ssem0, ssem1, ssem2, stsem0, stsem1):
    c = lax.axis_index("c")
    s = lax.axis_index("s")
    zeros16 = jnp.zeros((L,), jnp.float32)
    sems = (sem0, sem1, sem2)
    ssems = (ssem0, ssem1, ssem2)
    stsems = (stsem0, stsem1)
    stage = ((srce0_v, dste0_v), (srce1_v, dste1_v))

    def zero_zb(r, _):
        for j in range(HALF // L):
            zero_v[r, pl.ds(j * L, L)] = zeros16
        return _

    lax.fori_loop(0, 64, zero_zb, None)
    for j in range(SLAB // 64):
        pltpu.sync_copy(zero_v, agg_sh.at[pl.ds(s * SLAB + j * 64, 64)])

    plsc.subcore_barrier()

    ebase = s * EDGES_PER_TILE
    row_off = c * N_PAD

    def stage_chunk(cn, sync):
        se, de = stage[cn % 2]
        off = ebase + cn * CHUNK_C
        if sync:
            pltpu.sync_copy(ei_ref.at[pl.ds(off, CHUNK_C)], se)
            pltpu.sync_copy(ei_ref.at[pl.ds(N_EDGES + off, CHUNK_C)], de)
            return None
        return (pltpu.async_copy(ei_ref.at[pl.ds(off, CHUNK_C)], se,
                                 stsems[cn % 2]),
                pltpu.async_copy(ei_ref.at[pl.ds(N_EDGES + off, CHUNK_C)], de,
                                 stsems[cn % 2]))

    def main_phase(*bufs):
        srcvs = bufs[0:NBUF]
        dstvs = bufs[NBUF:2 * NBUF]
        rows = bufs[2 * NBUF:3 * NBUF]

        def prep(g, p):
            se, de = stage[(g // GROUPS_PER_CHUNK) % 2]
            goff = (g % GROUPS_PER_CHUNK) * K
            for u in range(K // L):
                sl = pl.ds(goff + u * L, L)
                srcvs[p][pl.ds(u * L, L)] = se[sl] + row_off
                dstvs[p][pl.ds(u * L, L)] = de[sl]

        stage_chunk(0, True)
        stdesc = [None] * N_CHUNKS
        stdesc[1] = stage_chunk(1, False)
        gdesc = [None] * NBUF
        sdesc = [None] * NBUF
        for w in range(NBUF - 1):
            prep(w, w)
            gdesc[w] = pltpu.async_copy(y2_ref.at[srcvs[w]], rows[w], sems[w])
        for g in range(N_GROUPS):
            p = g % NBUF
            gn = g + NBUF - 1
            if gn < N_GROUPS:
                q = gn % NBUF
                cn = gn // GROUPS_PER_CHUNK
                if gn % GROUPS_PER_CHUNK == 0:
                    for d in stdesc[cn]:
                        d.wait()
                    if cn + 1 < N_CHUNKS:
                        stdesc[cn + 1] = stage_chunk(cn + 1, False)
                if sdesc[q] is not None:
                    sdesc[q].wait()
                    sdesc[q] = None
                prep(gn, q)
                gdesc[q] = pltpu.async_copy(y2_ref.at[srcvs[q]], rows[q],
                                            sems[q])
            gdesc[p].wait()
            sdesc[p] = pltpu.make_async_copy(rows[p], agg_sh.at[dstvs[p]],
                                             ssems[p])
            sdesc[p].start(add=True)
        for p in range(NBUF):
            if sdesc[p] is not None:
                sdesc[p].wait()

    pl.run_scoped(
        main_phase,
        *([pltpu.VMEM((K,), jnp.int32)] * (2 * NBUF)
          + [pltpu.VMEM((K, HALF), jnp.float32)] * NBUF),
    )

    def t_phase(ndst_v, t_v):
        def zero_t(r, _):
            for j in range(HALF // L):
                t_v[r, pl.ds(j * L, L)] = zeros16
            return _

        lax.fori_loop(0, N_PAD // HALF, zero_t, None)
        pltpu.sync_copy(ndst_ref, ndst_v)

        def chunk_body(b, _):
            se, de = stage[0]
            pltpu.sync_copy(ei_ref.at[pl.ds(ebase + b * CHUNK_C, CHUNK_C)],
                            se)
            pltpu.sync_copy(
                ei_ref.at[pl.ds(N_EDGES + ebase + b * CHUNK_C, CHUNK_C)], de)

            units = CHUNK_C // L

            def sub(u, _):
                sl = pl.ds((2 * u + c) * L, L)
                sv = se[sl]
                dv = de[sl]
                vals = plsc.load_gather(
                    ndst_v, [lax.shift_right_logical(dv, 7),
                             jnp.bitwise_and(dv, 127)])
                plsc.addupdate_scatter(
                    t_v, [lax.shift_right_logical(sv, 7),
                          jnp.bitwise_and(sv, 127)], vals)
                return _

            lax.fori_loop(0, (units + 1) // 2 - c, sub, None)
            return _

        lax.fori_loop(0, N_CHUNKS, chunk_body, None)
        pltpu.sync_copy(t_v, tpart_ref.at[c, s])

    pl.run_scoped(t_phase,
                  pltpu.VMEM((N_PAD // HALF, HALF), jnp.float32),
                  pltpu.VMEM((N_PAD // HALF, HALF), jnp.float32))

    plsc.subcore_barrier()

    pltpu.sync_copy(agg_sh.at[pl.ds(s * SLAB, SLAB)],
                    agg_ref.at[pl.ds(c * N_PAD + s * SLAB, SLAB)])


def _edge_pass(y2, ei_flat, ndst_pad):
    return pl.kernel(
        _edge_body,
        out_type=[
            jax.ShapeDtypeStruct((2 * N_PAD, HALF), jnp.float32),
            jax.ShapeDtypeStruct((NC, NS, N_PAD // HALF, HALF), jnp.float32),
        ],
        mesh=plsc.VectorSubcoreMesh(core_axis_name="c", subcore_axis_name="s"),
        compiler_params=pltpu.CompilerParams(needs_layout_passes=False),
        scratch_types=[
            pltpu.VMEM_SHARED((N_PAD, HALF), jnp.float32),
            pltpu.VMEM((64, HALF), jnp.float32),
            pltpu.VMEM((CHUNK_C,), jnp.int32),
            pltpu.VMEM((CHUNK_C,), jnp.int32),
            pltpu.VMEM((CHUNK_C,), jnp.int32),
            pltpu.VMEM((CHUNK_C,), jnp.int32),
            pltpu.SemaphoreType.DMA,
            pltpu.SemaphoreType.DMA,
            pltpu.SemaphoreType.DMA,
            pltpu.SemaphoreType.DMA,
            pltpu.SemaphoreType.DMA,
            pltpu.SemaphoreType.DMA,
            pltpu.SemaphoreType.DMA,
            pltpu.SemaphoreType.DMA,
        ],
    )(y2, ei_flat, ndst_pad)


def _head_body(a0_ref, a1_ref, w1a_ref, w1b_ref, ndst_ref, csrc_ref,
               tpart_ref, b1_ref, w2_ref, b2_ref, out_ref, cacc):
    i = pl.program_id(0)

    @pl.when(i == 0)
    def _():
        cacc[...] = jnp.zeros((1, WIDTH), jnp.float32)

    z = jnp.dot(a0_ref[...].astype(jnp.bfloat16),
                w1a_ref[...].astype(jnp.bfloat16),
                preferred_element_type=jnp.float32)
    z += jnp.dot(a1_ref[...].astype(jnp.bfloat16),
                 w1b_ref[...].astype(jnp.bfloat16),
                 preferred_element_type=jnp.float32)
    z = z * ndst_ref[...] + b1_ref[...]
    h = jnp.maximum(z, 0.0)
    nsrc = lax.rsqrt(jnp.maximum(jnp.sum(csrc_ref[...], axis=0), 1.0))
    av = (nsrc * jnp.sum(tpart_ref[...], axis=0))[None, :]
    cacc[...] += jnp.dot(av, h, preferred_element_type=jnp.float32)

    @pl.when(i == GRID_N - 1)
    def _():
        out_ref[...] = (
            jnp.dot(cacc[...], w2_ref[...], preferred_element_type=jnp.float32)
            * (1.0 / N_NODES) + b2_ref[...])


def _head(agg, W1, ndst, cnt_part, tpart, b1, W2, b2):
    return pl.pallas_call(
        _head_body,
        grid=(GRID_N,),
        in_specs=[
            pl.BlockSpec((ROW_BLK, HALF), lambda i: (i, 0)),
            pl.BlockSpec((ROW_BLK, HALF), lambda i: (i + GRID_N, 0)),
            pl.BlockSpec((HALF, WIDTH), lambda i: (0, 0)),
            pl.BlockSpec((HALF, WIDTH), lambda i: (1, 0)),
            pl.BlockSpec((ROW_BLK, 1), lambda i: (i, 0)),
            pl.BlockSpec((NS, ROW_BLK), lambda i: (0, i)),
            pl.BlockSpec((NC * NS, ROW_BLK), lambda i: (0, i)),
            pl.BlockSpec((1, WIDTH), lambda i: (0, 0)),
            pl.BlockSpec((WIDTH, D_OUT), lambda i: (0, 0)),
            pl.BlockSpec((1, D_OUT), lambda i: (0, 0)),
        ],
        out_specs=pl.BlockSpec((1, D_OUT), lambda i: (0, 0)),
        out_shape=jax.ShapeDtypeStruct((1, D_OUT), jnp.float32),
        scratch_shapes=[pltpu.VMEM((1, WIDTH), jnp.float32)],
    )(agg, agg, W1, W1, ndst, cnt_part, tpart, b1, W2, b2)


@jax.jit
def kernel(x, edge_index, W1, b1, W2, b2):
    ei_flat = edge_index.reshape(-1)

    cnt_part = _degrees(ei_flat).reshape(NC * NS, N_PAD)
    xp = jnp.pad(x, ((0, N_PAD - N_NODES), (0, 0)))
    y, ndst = _scale(xp, cnt_part)
    y2 = y.reshape(2 * N_PAD, HALF)

    agg, tpart = _edge_pass(y2, ei_flat,
                            ndst.reshape(N_PAD // HALF, HALF))

    out = _head(agg, W1, ndst, cnt_part, tpart.reshape(NC * NS, N_PAD),
                b1.reshape(1, WIDTH), W2, b2.reshape(1, D_OUT))
    return out.reshape(D_OUT)


# speedup vs baseline: 1.0045x; 1.0045x over previous
import jax
import jax.numpy as jnp
from jax import lax
from jax.experimental import pallas as pl
from jax.experimental.pallas import tpu as pltpu
from jax.experimental.pallas import tpu_sc as plsc

N_NODES = 10000
N_PAD = 10240
N_EDGES = 160000
D_IN = 256
WIDTH = 1024
D_OUT = 256
HALF = 128

NC = 2
NS = 16
L = 16

EDGES_PER_TILE = N_EDGES // NS
K = 80
CHUNK_C = 2000
SLAB = N_PAD // NS

ROW_BLK = 1024
GRID_N = N_PAD // ROW_BLK


def _degrees_body(ei_ref, cnt_out_ref, cnt_v, idx_v, sem):
    c = lax.axis_index("c")
    s = lax.axis_index("s")
    zeros16 = jnp.zeros((L,), jnp.float32)
    ones16 = jnp.ones((L,), jnp.float32)

    base = c * N_EDGES + s * EDGES_PER_TILE
    idesc = pltpu.async_copy(ei_ref.at[pl.ds(base, EDGES_PER_TILE)], idx_v,
                             sem)

    def zero_body(r, _):
        for j in range(HALF // L):
            cnt_v[r, pl.ds(j * L, L)] = zeros16
        return _

    lax.fori_loop(0, N_PAD // HALF, zero_body, None)
    idesc.wait()

    def scat_body(u, _):
        for k in range(K // L):
            idx = idx_v[pl.ds(u * K + k * L, L)]
            row = lax.shift_right_logical(idx, 7)
            col = jnp.bitwise_and(idx, 127)
            plsc.addupdate_scatter(cnt_v, [row, col], ones16)
        return _

    lax.fori_loop(0, EDGES_PER_TILE // K, scat_body, None)
    pltpu.sync_copy(cnt_v, cnt_out_ref.at[c, s])


def _degrees(ei_flat):
    return pl.kernel(
        _degrees_body,
        out_type=jax.ShapeDtypeStruct((NC, NS, N_PAD // HALF, HALF),
                                      jnp.float32),
        mesh=plsc.VectorSubcoreMesh(core_axis_name="c", subcore_axis_name="s"),
        compiler_params=pltpu.CompilerParams(needs_layout_passes=False),
        scratch_types=[
            pltpu.VMEM((N_PAD // HALF, HALF), jnp.float32),
            pltpu.VMEM((EDGES_PER_TILE,), jnp.int32),
            pltpu.SemaphoreType.DMA,
        ],
    )(ei_flat)


def _scale_body(x_ref, csrc_ref, cdst_ref, y_ref, ndst_ref):
    csum = jnp.sum(csrc_ref[...], axis=0)
    dsum = jnp.sum(cdst_ref[...], axis=0)
    nsrc = lax.rsqrt(jnp.maximum(csum, 1.0))
    ndst_ref[...] = lax.rsqrt(jnp.maximum(dsum, 1.0))[:, None]
    y_ref[0] = x_ref[:, :HALF] * nsrc[:, None]
    y_ref[1] = x_ref[:, HALF:] * nsrc[:, None]


def _scale(xp, cnt_part):
    return pl.pallas_call(
        _scale_body,
        grid=(GRID_N,),
        in_specs=[
            pl.BlockSpec((ROW_BLK, D_IN), lambda i: (i, 0)),
            pl.BlockSpec((NS, ROW_BLK), lambda i: (0, i)),
            pl.BlockSpec((NS, ROW_BLK), lambda i: (1, i)),
        ],
        out_specs=[
            pl.BlockSpec((2, ROW_BLK, HALF), lambda i: (0, i, 0)),
            pl.BlockSpec((ROW_BLK, 1), lambda i: (i, 0)),
        ],
        out_shape=[
            jax.ShapeDtypeStruct((2, N_PAD, HALF), jnp.float32),
            jax.ShapeDtypeStruct((N_PAD, 1), jnp.float32),
        ],
    )(xp, cnt_part, cnt_part)


NBUF = 3
N_CHUNKS = EDGES_PER_TILE // CHUNK_C
GROUPS_PER_CHUNK = CHUNK_C // K
N_GROUPS = EDGES_PER_TILE // K


def _edge_body(y2_ref, ei_ref, ndst_ref, agg_ref, tpart_ref,
               agg_sh, zero_v, srce0_v, dste0_v, srce1_v, dste1_v,
               sem0, sem1, sem2, ssem0, ssem1, ssem2, stsem0, stsem1):
    c = lax.axis_index("c")
    s = lax.axis_index("s")
    zeros16 = jnp.zeros((L,), jnp.float32)
    sems = (sem0, sem1, sem2)
    ssems = (ssem0, ssem1, ssem2)
    stsems = (stsem0, stsem1)
    stage = ((srce0_v, dste0_v), (srce1_v, dste1_v))

    def zero_zb(r, _):
        for j in range(HALF // L):
            zero_v[r, pl.ds(j * L, L)] = zeros16
        return _

    lax.fori_loop(0, 64, zero_zb, None)
    for j in range(SLAB // 64):
        pltpu.sync_copy(zero_v, agg_sh.at[pl.ds(s * SLAB + j * 64, 64)])

    plsc.subcore_barrier()

    ebase = s * EDGES_PER_TILE
    row_off = c * N_PAD

    def stage_chunk(cn, sync):
        se, de = stage[cn % 2]
        off = ebase + cn * CHUNK_C
        if sync:
            pltpu.sync_copy(ei_ref.at[pl.ds(off, CHUNK_C)], se)
            pltpu.sync_copy(ei_ref.at[pl.ds(N_EDGES + off, CHUNK_C)], de)
            return None
        return (pltpu.async_copy(ei_ref.at[pl.ds(off, CHUNK_C)], se,
                                 stsems[cn % 2]),
                pltpu.async_copy(ei_ref.at[pl.ds(N_EDGES + off, CHUNK_C)], de,
                                 stsems[cn % 2]))

    def main_phase(*bufs):
        srcvs = bufs[0:NBUF]
        dstvs = bufs[NBUF:2 * NBUF]
        rows = bufs[2 * NBUF:3 * NBUF]

        def prep(g, p):
            se, de = stage[(g // GROUPS_PER_CHUNK) % 2]
            goff = (g % GROUPS_PER_CHUNK) * K
            for u in range(K // L):
                sl = pl.ds(goff + u * L, L)
                srcvs[p][pl.ds(u * L, L)] = se[sl] + row_off
                dstvs[p][pl.ds(u * L, L)] = de[sl]

        stage_chunk(0, True)
        stdesc = [None] * N_CHUNKS
        stdesc[1] = stage_chunk(1, False)
        gdesc = [None] * NBUF
        sdesc = [None] * NBUF
        for w in range(NBUF - 1):
            prep(w, w)
            gdesc[w] = pltpu.async_copy(y2_ref.at[srcvs[w]], rows[w], sems[w])
        for g in range(N_GROUPS):
            p = g % NBUF
            gn = g + NBUF - 1
            if gn < N_GROUPS:
                q = gn % NBUF
                cn = gn // GROUPS_PER_CHUNK
                if gn % GROUPS_PER_CHUNK == 0:
                    for d in stdesc[cn]:
                        d.wait()
                    if cn + 1 < N_CHUNKS:
                        stdesc[cn + 1] = stage_chunk(cn + 1, False)
                if sdesc[q] is not None:
                    sdesc[q].wait()
                    sdesc[q] = None
                prep(gn, q)
                gdesc[q] = pltpu.async_copy(y2_ref.at[srcvs[q]], rows[q],
                                            sems[q])
            gdesc[p].wait()
            sdesc[p] = pltpu.make_async_copy(rows[p], agg_sh.at[dstvs[p]],
                                             ssems[p])
            sdesc[p].start(add=True)
        for p in range(NBUF):
            if sdesc[p] is not None:
                sdesc[p].wait()

    pl.run_scoped(
        main_phase,
        *([pltpu.VMEM((K,), jnp.int32)] * (2 * NBUF)
          + [pltpu.VMEM((K, HALF), jnp.float32)] * NBUF),
    )

    def t_phase(ndst_v, t_v):
        def zero_t(r, _):
            for j in range(HALF // L):
                t_v[r, pl.ds(j * L, L)] = zeros16
            return _

        lax.fori_loop(0, N_PAD // HALF, zero_t, None)
        pltpu.sync_copy(ndst_ref, ndst_v)

        def chunk_body(b, _):
            se, de = stage[0]
            pltpu.sync_copy(ei_ref.at[pl.ds(ebase + b * CHUNK_C, CHUNK_C)],
                            se)
            pltpu.sync_copy(
                ei_ref.at[pl.ds(N_EDGES + ebase + b * CHUNK_C, CHUNK_C)], de)

            units = CHUNK_C // L

            def sub(u, _):
                sl = pl.ds((2 * u + c) * L, L)
                sv = se[sl]
                dv = de[sl]
                vals = plsc.load_gather(
                    ndst_v, [lax.shift_right_logical(dv, 7),
                             jnp.bitwise_and(dv, 127)])
                plsc.addupdate_scatter(
                    t_v, [lax.shift_right_logical(sv, 7),
                          jnp.bitwise_and(sv, 127)], vals)
                return _

            lax.fori_loop(0, (units + 1) // 2 - c, sub, None)
            return _

        lax.fori_loop(0, N_CHUNKS, chunk_body, None)
        pltpu.sync_copy(t_v, tpart_ref.at[c, s])

    pl.run_scoped(t_phase,
                  pltpu.VMEM((N_PAD // HALF, HALF), jnp.float32),
                  pltpu.VMEM((N_PAD // HALF, HALF), jnp.float32))

    plsc.subcore_barrier()

    pltpu.sync_copy(agg_sh.at[pl.ds(s * SLAB, SLAB)],
                    agg_ref.at[pl.ds(c * N_PAD + s * SLAB, SLAB)])


def _edge_pass(y2, ei_flat, ndst_pad):
    return pl.kernel(
        _edge_body,
        out_type=[
            jax.ShapeDtypeStruct((2 * N_PAD, HALF), jnp.float32),
            jax.ShapeDtypeStruct((NC, NS, N_PAD // HALF, HALF), jnp.float32),
        ],
        mesh=plsc.VectorSubcoreMesh(core_axis_name="c", subcore_axis_name="s"),
        compiler_params=pltpu.CompilerParams(needs_layout_passes=False),
        scratch_types=[
            pltpu.VMEM_SHARED((N_PAD, HALF), jnp.float32),
            pltpu.VMEM((64, HALF), jnp.float32),
            pltpu.VMEM((CHUNK_C,), jnp.int32),
            pltpu.VMEM((CHUNK_C,), jnp.int32),
            pltpu.VMEM((CHUNK_C,), jnp.int32),
            pltpu.VMEM((CHUNK_C,), jnp.int32),
            pltpu.SemaphoreType.DMA,
            pltpu.SemaphoreType.DMA,
            pltpu.SemaphoreType.DMA,
            pltpu.SemaphoreType.DMA,
            pltpu.SemaphoreType.DMA,
            pltpu.SemaphoreType.DMA,
            pltpu.SemaphoreType.DMA,
            pltpu.SemaphoreType.DMA,
        ],
    )(y2, ei_flat, ndst_pad)


def _head_body(a0_ref, a1_ref, w1a_ref, w1b_ref, ndst_ref, csrc_ref,
               tpart_ref, b1_ref, w2_ref, b2_ref, out_ref, cacc):
    i = pl.program_id(0)

    @pl.when(i == 0)
    def _():
        cacc[...] = jnp.zeros((1, WIDTH), jnp.float32)

    z = jnp.dot(a0_ref[...].astype(jnp.bfloat16),
                w1a_ref[...].astype(jnp.bfloat16),
                preferred_element_type=jnp.float32)
    z += jnp.dot(a1_ref[...].astype(jnp.bfloat16),
                 w1b_ref[...].astype(jnp.bfloat16),
                 preferred_element_type=jnp.float32)
    z = z * ndst_ref[...] + b1_ref[...]
    h = jnp.maximum(z, 0.0)
    nsrc = lax.rsqrt(jnp.maximum(jnp.sum(csrc_ref[...], axis=0), 1.0))
    av = (nsrc * jnp.sum(tpart_ref[...], axis=0))[None, :]
    cacc[...] += jnp.dot(av, h, preferred_element_type=jnp.float32)

    @pl.when(i == GRID_N - 1)
    def _():
        out_ref[...] = (
            jnp.dot(cacc[...], w2_ref[...], preferred_element_type=jnp.float32)
            * (1.0 / N_NODES) + b2_ref[...])


def _head(agg, W1, ndst, cnt_part, tpart, b1, W2, b2):
    return pl.pallas_call(
        _head_body,
        grid=(GRID_N,),
        in_specs=[
            pl.BlockSpec((ROW_BLK, HALF), lambda i: (i, 0)),
            pl.BlockSpec((ROW_BLK, HALF), lambda i: (i + GRID_N, 0)),
            pl.BlockSpec((HALF, WIDTH), lambda i: (0, 0)),
            pl.BlockSpec((HALF, WIDTH), lambda i: (1, 0)),
            pl.BlockSpec((ROW_BLK, 1), lambda i: (i, 0)),
            pl.BlockSpec((NS, ROW_BLK), lambda i: (0, i)),
            pl.BlockSpec((NC * NS, ROW_BLK), lambda i: (0, i)),
            pl.BlockSpec((1, WIDTH), lambda i: (0, 0)),
            pl.BlockSpec((WIDTH, D_OUT), lambda i: (0, 0)),
            pl.BlockSpec((1, D_OUT), lambda i: (0, 0)),
        ],
        out_specs=pl.BlockSpec((1, D_OUT), lambda i: (0, 0)),
        out_shape=jax.ShapeDtypeStruct((1, D_OUT), jnp.float32),
        scratch_shapes=[pltpu.VMEM((1, WIDTH), jnp.float32)],
    )(agg, agg, W1, W1, ndst, cnt_part, tpart, b1, W2, b2)


@jax.jit
def kernel(x, edge_index, W1, b1, W2, b2):
    ei_flat = edge_index.reshape(-1)

    cnt_part = _degrees(ei_flat).reshape(NC * NS, N_PAD)
    xp = jnp.pad(x, ((0, N_PAD - N_NODES), (0, 0)))
    y, ndst = _scale(xp, cnt_part)
    y2 = y.reshape(2 * N_PAD, HALF)

    agg, tpart = _edge_pass(y2, ei_flat,
                            ndst.reshape(N_PAD // HALF, HALF))

    out = _head(agg, W1, ndst, cnt_part, tpart.reshape(NC * NS, N_PAD),
                b1.reshape(1, WIDTH), W2, b2.reshape(1, D_OUT))
    return out.reshape(D_OUT)
